# baseline (speedup 1.0000x reference)
import functools

import jax
import jax.numpy as jnp
from jax import lax
from jax.experimental import pallas as pl
from jax.experimental.pallas import tpu as pltpu
from jax.experimental.pallas import tpu_sc as plsc

B = 8192
K = 32
D = 256
E = 256
L = 16
VPR = D // L
NC, NS = 2, 16
NW = NC * NS
BPW = B // NW
CH_N = 4
CR = CH_N * K
NCH = BPW // CH_N
SH = 128
BM = 8192


def _sc_gather_sum(nodes_hbm, neighs_hbm, feat_hbm, self_out, agg_out,
                   sidx_v, nidx_v, sbuf, nbuf0, nbuf1, abuf0, abuf1,
                   ssem, nsem0, nsem1, asem0, asem1):
    wid = lax.axis_index("s") * NC + lax.axis_index("c")
    base = wid * BPW
    nbase = base * K

    pltpu.sync_copy(nodes_hbm.at[pl.ds(base, BPW)], sidx_v)
    pltpu.sync_copy(neighs_hbm.at[pl.ds(nbase, BPW * K)], nidx_v)

    pltpu.async_copy(feat_hbm.at[sidx_v.at[pl.ds(0, SH)]], sbuf, ssem)

    pltpu.async_copy(feat_hbm.at[nidx_v.at[pl.ds(0, CR)]], nbuf0, nsem0)
    pltpu.async_copy(feat_hbm.at[nidx_v.at[pl.ds(CR, CR)]], nbuf1, nsem1)

    def reduce_chunk(buf, abuf):
        for b in range(CH_N):
            def red(r, accs):
                return tuple(
                    functools.reduce(
                        lambda a, j: a + buf[b * K + 4 * r + j,
                                             pl.ds(v * L, L)],
                        range(4), accs[v])
                    for v in range(VPR))
            accs = lax.fori_loop(
                0, K // 4, red,
                tuple(jnp.zeros((L,), jnp.float32) for _ in range(VPR)))
            for v in range(VPR):
                abuf[b, pl.ds(v * L, L)] = accs[v]

    def step(c, buf, sem, abuf, asem):
        pltpu.make_async_copy(
            feat_hbm.at[nidx_v.at[pl.ds(c * CR, CR)]], buf, sem).wait()

        @pl.when(c >= 2)
        def _():
            pltpu.make_async_copy(
                abuf, agg_out.at[pl.ds(base, CH_N)], asem).wait()

        reduce_chunk(buf, abuf)
        pltpu.async_copy(abuf, agg_out.at[pl.ds(base + c * CH_N, CH_N)],
                         asem)

        @pl.when(c < NCH - 2)
        def _():
            pltpu.async_copy(
                feat_hbm.at[nidx_v.at[pl.ds((c + 2) * CR, CR)]], buf, sem)

    def body(i, carry):
        step(2 * i, nbuf0, nsem0, abuf0, asem0)

        @pl.when(i == NCH // 4)
        def _():
            pltpu.make_async_copy(
                feat_hbm.at[sidx_v.at[pl.ds(0, SH)]], sbuf, ssem).wait()
            pltpu.sync_copy(sbuf, self_out.at[pl.ds(base, SH)])
            pltpu.async_copy(feat_hbm.at[sidx_v.at[pl.ds(SH, SH)]],
                             sbuf, ssem)

        step(2 * i + 1, nbuf1, nsem1, abuf1, asem1)
        return carry

    lax.fori_loop(0, NCH // 2, body, 0)

    pltpu.make_async_copy(abuf0, agg_out.at[pl.ds(base, CH_N)],
                          asem0).wait()
    pltpu.make_async_copy(abuf1, agg_out.at[pl.ds(base, CH_N)],
                          asem1).wait()
    pltpu.make_async_copy(
        feat_hbm.at[sidx_v.at[pl.ds(SH, SH)]], sbuf, ssem).wait()
    pltpu.sync_copy(sbuf, self_out.at[pl.ds(base + SH, SH)])


_sc_call = functools.partial(
    pl.kernel,
    out_type=[jax.ShapeDtypeStruct((B, D), jnp.float32),
              jax.ShapeDtypeStruct((B, D), jnp.float32)],
    mesh=plsc.VectorSubcoreMesh(core_axis_name="c", subcore_axis_name="s"),
    scratch_types=[
        pltpu.VMEM((BPW,), jnp.int32),
        pltpu.VMEM((BPW * K,), jnp.int32),
        pltpu.VMEM((SH, D), jnp.float32),
        pltpu.VMEM((CR, D), jnp.float32),
        pltpu.VMEM((CR, D), jnp.float32),
        pltpu.VMEM((CH_N, D), jnp.float32),
        pltpu.VMEM((CH_N, D), jnp.float32),
        pltpu.SemaphoreType.DMA,
        pltpu.SemaphoreType.DMA,
        pltpu.SemaphoreType.DMA,
        pltpu.SemaphoreType.DMA,
        pltpu.SemaphoreType.DMA,
    ],
)(_sc_gather_sum)


def _mm_kernel(x1_ref, x2_ref, w1_ref, w2_ref, o_ref):
    acc = jnp.dot(x1_ref[...], w1_ref[...], preferred_element_type=jnp.float32)
    acc = acc + jnp.dot(x2_ref[...], w2_ref[...],
                        preferred_element_type=jnp.float32) * (1.0 / K)
    o_ref[...] = jnp.maximum(acc, 0.0)


@jax.jit
def kernel(nodes, to_neighs, features, weight):
    nodes_i = nodes.astype(jnp.int32)
    neighs_flat = to_neighs.reshape(-1).astype(jnp.int32)
    self_feats, agg_sums = _sc_call(nodes_i, neighs_flat, features)
    return pl.pallas_call(
        _mm_kernel,
        grid=(B // BM,),
        in_specs=[
            pl.BlockSpec((BM, D), lambda i: (i, 0)),
            pl.BlockSpec((BM, D), lambda i: (i, 0)),
            pl.BlockSpec((D, E), lambda i: (0, 0)),
            pl.BlockSpec((D, E), lambda i: (1, 0)),
        ],
        out_specs=pl.BlockSpec((BM, E), lambda i: (i, 0)),
        out_shape=jax.ShapeDtypeStruct((B, E), jnp.float32),
    )(self_feats, agg_sums, weight, weight)


# speedup vs baseline: 1.0154x; 1.0154x over previous
import functools

import jax
import jax.numpy as jnp
from jax import lax
from jax.experimental import pallas as pl
from jax.experimental.pallas import tpu as pltpu
from jax.experimental.pallas import tpu_sc as plsc

B = 8192
K = 32
D = 256
E = 256
L = 16
VPR = D // L
NC, NS = 2, 16
NW = NC * NS
BPW = B // NW
CH_N = 4
CR = CH_N * K
NCH = BPW // CH_N
SH = 128
BM = 4096


def _sc_gather_sum(nodes_hbm, neighs_hbm, feat_hbm, self_out, agg_out,
                   sidx_v, nidx_v, sbuf, nbuf0, nbuf1, abuf0, abuf1,
                   ssem, nsem0, nsem1, asem0, asem1):
    wid = lax.axis_index("s") * NC + lax.axis_index("c")
    base = wid * BPW
    nbase = base * K

    pltpu.sync_copy(nodes_hbm.at[pl.ds(base, BPW)], sidx_v)
    pltpu.sync_copy(neighs_hbm.at[pl.ds(nbase, BPW * K)], nidx_v)

    pltpu.async_copy(feat_hbm.at[sidx_v.at[pl.ds(0, SH)]], sbuf, ssem)

    pltpu.async_copy(feat_hbm.at[nidx_v.at[pl.ds(0, CR)]], nbuf0, nsem0)
    pltpu.async_copy(feat_hbm.at[nidx_v.at[pl.ds(CR, CR)]], nbuf1, nsem1)

    def reduce_chunk(buf, abuf):
        for b in range(CH_N):
            def red(r, accs):
                return tuple(
                    functools.reduce(
                        lambda a, j: a + buf[b * K + 4 * r + j,
                                             pl.ds(v * L, L)],
                        range(4), accs[v])
                    for v in range(VPR))
            accs = lax.fori_loop(
                0, K // 4, red,
                tuple(jnp.zeros((L,), jnp.float32) for _ in range(VPR)))
            for v in range(VPR):
                abuf[b, pl.ds(v * L, L)] = accs[v]

    def step(c, buf, sem, abuf, asem):
        pltpu.make_async_copy(
            feat_hbm.at[nidx_v.at[pl.ds(c * CR, CR)]], buf, sem).wait()

        @pl.when(c >= 2)
        def _():
            pltpu.make_async_copy(
                abuf, agg_out.at[pl.ds(base, CH_N)], asem).wait()

        reduce_chunk(buf, abuf)
        pltpu.async_copy(abuf, agg_out.at[pl.ds(base + c * CH_N, CH_N)],
                         asem)

        @pl.when(c < NCH - 2)
        def _():
            pltpu.async_copy(
                feat_hbm.at[nidx_v.at[pl.ds((c + 2) * CR, CR)]], buf, sem)

    def body(i, carry):
        step(2 * i, nbuf0, nsem0, abuf0, asem0)

        @pl.when(i == NCH // 4)
        def _():
            pltpu.make_async_copy(
                feat_hbm.at[sidx_v.at[pl.ds(0, SH)]], sbuf, ssem).wait()
            pltpu.sync_copy(sbuf, self_out.at[pl.ds(base, SH)])
            pltpu.async_copy(feat_hbm.at[sidx_v.at[pl.ds(SH, SH)]],
                             sbuf, ssem)

        step(2 * i + 1, nbuf1, nsem1, abuf1, asem1)
        return carry

    lax.fori_loop(0, NCH // 2, body, 0)

    pltpu.make_async_copy(abuf0, agg_out.at[pl.ds(base, CH_N)],
                          asem0).wait()
    pltpu.make_async_copy(abuf1, agg_out.at[pl.ds(base, CH_N)],
                          asem1).wait()
    pltpu.make_async_copy(
        feat_hbm.at[sidx_v.at[pl.ds(SH, SH)]], sbuf, ssem).wait()
    pltpu.sync_copy(sbuf, self_out.at[pl.ds(base + SH, SH)])


_sc_call = functools.partial(
    pl.kernel,
    out_type=[jax.ShapeDtypeStruct((B, D), jnp.float32),
              jax.ShapeDtypeStruct((B, D), jnp.float32)],
    mesh=plsc.VectorSubcoreMesh(core_axis_name="c", subcore_axis_name="s"),
    scratch_types=[
        pltpu.VMEM((BPW,), jnp.int32),
        pltpu.VMEM((BPW * K,), jnp.int32),
        pltpu.VMEM((SH, D), jnp.float32),
        pltpu.VMEM((CR, D), jnp.float32),
        pltpu.VMEM((CR, D), jnp.float32),
        pltpu.VMEM((CH_N, D), jnp.float32),
        pltpu.VMEM((CH_N, D), jnp.float32),
        pltpu.SemaphoreType.DMA,
        pltpu.SemaphoreType.DMA,
        pltpu.SemaphoreType.DMA,
        pltpu.SemaphoreType.DMA,
        pltpu.SemaphoreType.DMA,
    ],
)(_sc_gather_sum)


def _mm_kernel(x1_ref, x2_ref, w1_ref, w2_ref, o_ref):
    acc = jnp.dot(x1_ref[...], w1_ref[...], preferred_element_type=jnp.float32)
    acc = acc + jnp.dot(x2_ref[...], w2_ref[...],
                        preferred_element_type=jnp.float32) * (1.0 / K)
    o_ref[...] = jnp.maximum(acc, 0.0)


@jax.jit
def kernel(nodes, to_neighs, features, weight):
    nodes_i = nodes.astype(jnp.int32)
    neighs_flat = to_neighs.reshape(-1).astype(jnp.int32)
    self_feats, agg_sums = _sc_call(nodes_i, neighs_flat, features)
    return pl.pallas_call(
        _mm_kernel,
        grid=(B // BM,),
        in_specs=[
            pl.BlockSpec((BM, D), lambda i: (i, 0)),
            pl.BlockSpec((BM, D), lambda i: (i, 0)),
            pl.BlockSpec((D, E), lambda i: (0, 0)),
            pl.BlockSpec((D, E), lambda i: (1, 0)),
        ],
        out_specs=pl.BlockSpec((BM, E), lambda i: (i, 0)),
        out_shape=jax.ShapeDtypeStruct((B, E), jnp.float32),
    )(self_feats, agg_sums, weight, weight)
